# Initial kernel scaffold; baseline (speedup 1.0000x reference)
import jax
import jax.numpy as jnp
from jax.experimental import pallas as pl


def kernel(x, t1, b1, t2, b2, w1, fb1, w2, fb2):
    raise NotImplementedError("write your pallas kernel here")


# speedup vs baseline: 3.6105x; 3.6105x over previous
import functools

import jax
import jax.numpy as jnp
from jax.experimental import pallas as pl
from jax.experimental.pallas import tpu as pltpu

TB = 128
VMEM_LIMIT_BYTES = 96 * 1024 * 1024


def _fwd_kernel(x_ref, t1_ref, b1_ref, t2_ref, b2_ref,
                w1_ref, fb1_ref, w2_ref, fb2_ref,
                o_ref, a1_scr):
    f32 = jnp.float32
    bf16 = jnp.bfloat16

    lhs = jnp.concatenate(
        [x_ref[kh:kh + 32].reshape(32 * TB, 128) for kh in range(3)], axis=1)
    acc = jnp.dot(lhs, t1_ref[...], preferred_element_type=f32)
    y = jnp.maximum(acc + b1_ref[...], 0.0)
    yh = jnp.maximum(y[:, :256], y[:, 256:]).reshape(16, 2 * TB, 256)
    a1_scr[1:17] = jnp.maximum(yh[:, :TB], yh[:, TB:]).astype(bf16)
    zpad = jnp.zeros((1, TB, 256), bf16)
    a1_scr[0:1] = zpad
    a1_scr[17:18] = zpad

    acc2 = jnp.dot(a1_scr[0:16].reshape(16 * TB, 256), t2_ref[0],
                   preferred_element_type=f32)
    for kh in (1, 2):
        acc2 = acc2 + jnp.dot(a1_scr[kh:kh + 16].reshape(16 * TB, 256),
                              t2_ref[kh], preferred_element_type=f32)
    y2 = jnp.maximum(acc2 + b2_ref[...], 0.0)
    yh2 = jnp.maximum(y2[:, :128], y2[:, 128:]).reshape(8, 2 * TB, 128)
    pooled = jnp.maximum(yh2[:, :TB], yh2[:, TB:]).astype(bf16)

    feats = jnp.concatenate([pooled[i] for i in range(8)], axis=1)
    h1 = jnp.maximum(
        jnp.dot(feats, w1_ref[...], preferred_element_type=f32) + fb1_ref[...],
        0.0)
    out = jnp.dot(h1.astype(bf16), w2_ref[...],
                  preferred_element_type=f32) + fb2_ref[...]
    o_ref[...] = out.astype(o_ref.dtype)


@jax.jit
def _forward(x_nchw, t1, b1, t2, b2, w1, fb1, w2, fb2):
    B, Cin, H, W = x_nchw.shape
    Bp = -(-B // TB) * TB

    xh = jnp.transpose(x_nchw, (2, 0, 3, 1)).reshape(H, B, W * Cin)
    xh = jnp.pad(xh, ((1, 1), (0, Bp - B), (0, 32))).astype(jnp.bfloat16)

    t1p = jnp.pad(t1, ((0, 0), (0, 32), (0, 0))).reshape(384, 512)

    out = pl.pallas_call(
        _fwd_kernel,
        out_shape=jax.ShapeDtypeStruct((Bp, 128), jnp.float32),
        grid=(Bp // TB,),
        in_specs=[
            pl.BlockSpec((H + 2, TB, 128), lambda i: (0, i, 0)),
            pl.BlockSpec((384, 512), lambda i: (0, 0)),
            pl.BlockSpec((1, 512), lambda i: (0, 0)),
            pl.BlockSpec((3, 256, 256), lambda i: (0, 0, 0)),
            pl.BlockSpec((1, 256), lambda i: (0, 0)),
            pl.BlockSpec((1024, 256), lambda i: (0, 0)),
            pl.BlockSpec((1, 256), lambda i: (0, 0)),
            pl.BlockSpec((256, 128), lambda i: (0, 0)),
            pl.BlockSpec((1, 128), lambda i: (0, 0)),
        ],
        out_specs=pl.BlockSpec((TB, 128), lambda i: (i, 0)),
        scratch_shapes=[pltpu.VMEM((18, TB, 256), jnp.bfloat16)],
        compiler_params=pltpu.CompilerParams(
            dimension_semantics=("parallel",),
            vmem_limit_bytes=VMEM_LIMIT_BYTES),
    )(xh, t1p.astype(jnp.bfloat16), b1, t2.astype(jnp.bfloat16), b2,
      w1.astype(jnp.bfloat16), fb1, w2.astype(jnp.bfloat16), fb2)
    return out[:B, :10]


def kernel(x, t1, b1, t2, b2, w1, fb1, w2, fb2):
    return _forward(x, t1, b1, t2, b2, w1, fb1, w2, fb2)
